# baseline (speedup 1.0000x reference)
import functools

import jax
import jax.numpy as jnp
from jax import lax
from jax.experimental import pallas as pl
from jax.experimental.pallas import tpu as pltpu
from jax.experimental.pallas import tpu_sc as plsc

NUM_FIELDS = 8
VOCAB = 100000
EMB = 64
PROJ = 512
DTOT = NUM_FIELDS * EMB

NC, NS = 2, 16
NW = NC * NS
CHUNK = 128
NBUF = 5


@functools.lru_cache(maxsize=None)
def _make_gather(n_rows: int, slice_chunk_base: int):
    rows_per_w = n_rows // NW
    chunks_per_w = rows_per_w // CHUNK
    mesh = plsc.VectorSubcoreMesh(core_axis_name="c", subcore_axis_name="s")

    assert chunks_per_w % NBUF == 0
    n_groups = chunks_per_w // NBUF

    @functools.partial(
        pl.kernel,
        mesh=mesh,
        compiler_params=pltpu.CompilerParams(use_tc_tiling_on_sc=False),
        out_type=jax.ShapeDtypeStruct((n_rows, EMB), jnp.bfloat16),
        scratch_types=(
            [pltpu.VMEM((chunks_per_w, CHUNK), jnp.int32)]
            + [pltpu.VMEM((CHUNK, EMB), jnp.bfloat16)] * NBUF
            + [pltpu.SemaphoreType.DMA] * (2 * NBUF)
        ),
    )
    def gather(table_hbm, idx_hbm, out_hbm, idx_v, *scratch):
        wid = lax.axis_index("s") * NC + lax.axis_index("c")
        cbase = slice_chunk_base + wid * chunks_per_w
        rbase = wid * rows_per_w
        bufs = list(scratch[:NBUF])
        gsems = list(scratch[NBUF:2 * NBUF])
        wsems = list(scratch[2 * NBUF:])
        tbl = table_hbm
        pltpu.sync_copy(idx_hbm.at[pl.ds(cbase, chunks_per_w)], idx_v)

        def gather_cp(j, b):
            return pltpu.make_async_copy(
                tbl.at[idx_v.at[j]], bufs[b], gsems[b]
            )

        def wb_cp(j, b):
            return pltpu.make_async_copy(
                bufs[b], out_hbm.at[pl.ds(rbase + j * CHUNK, CHUNK)], wsems[b]
            )

        for b in range(NBUF):
            gather_cp(b, b).start()

        def group(g, _):
            for b in range(NBUF):
                j = g * NBUF + b
                gather_cp(j, b).wait()
                wb_cp(j, b).start()

                @pl.when(g < n_groups - 1)
                def _():
                    wb_cp(j, b).wait()
                    gather_cp(j + NBUF, b).start()
            return _

        lax.fori_loop(0, n_groups, group, None)

        for b in range(NBUF):
            wb_cp(chunks_per_w - NBUF + b, b).wait()

    return gather


VB = 2048
NVB = VOCAB // VB
VMAIN = NVB * VB
VTAIL = VOCAB - VMAIN
HTAIL = VTAIL // 2
ROWS_F = VMAIN // 2
MAIN_ROWS = NUM_FIELDS * ROWS_F
TAIL_SLOT = 1024
TROWS = MAIN_ROWS + NUM_FIELDS * TAIL_SLOT


def _mxu_t(x, eye):
    return jax.lax.dot_general(
        x, eye, (((0,), (0,)), ((), ())),
        preferred_element_type=jnp.float32,
    )


def _tr_main_body(x_ref, eye_ref, o_ref):
    x = x_ref[0]
    eye = eye_ref[...]
    o_ref[...] = jnp.concatenate(
        [_mxu_t(x[:, :1024], eye), _mxu_t(x[:, 1024:], eye)], axis=1
    ).astype(jnp.bfloat16)


def _tr_tail_body(xa_ref, xb_ref, eye_ref, _prev_ref, o_ref):
    eye = eye_ref[...]
    t = jnp.concatenate([_mxu_t(xa_ref[0], eye), _mxu_t(xb_ref[0], eye)],
                        axis=1).astype(jnp.bfloat16)
    o_ref[...] = jnp.concatenate(
        [t, jnp.zeros((TAIL_SLOT - HTAIL, 2 * EMB), jnp.bfloat16)], axis=0
    )


@functools.lru_cache(maxsize=None)
def _make_transpose_main():
    return pl.pallas_call(
        _tr_main_body,
        grid=(NUM_FIELDS, NVB),
        in_specs=[
            pl.BlockSpec((1, EMB, VB), lambda f, v: (f, 0, v)),
            pl.BlockSpec((EMB, EMB), lambda f, v: (0, 0)),
        ],
        out_specs=pl.BlockSpec(
            (VB // 2, 2 * EMB), lambda f, v: (f * NVB + v, 0)
        ),
        out_shape=jax.ShapeDtypeStruct((TROWS, 2 * EMB), jnp.bfloat16),
    )


@functools.lru_cache(maxsize=None)
def _make_transpose_tail():
    return pl.pallas_call(
        _tr_tail_body,
        grid=(NUM_FIELDS,),
        in_specs=[
            pl.BlockSpec((1, EMB, HTAIL), lambda f: (f, 0, 0)),
            pl.BlockSpec((1, EMB, HTAIL), lambda f: (f, 0, 0)),
            pl.BlockSpec((EMB, EMB), lambda f: (0, 0)),
            pl.BlockSpec(memory_space=pl.ANY),
        ],
        out_specs=pl.BlockSpec(
            (TAIL_SLOT, 2 * EMB), lambda f: (MAIN_ROWS // TAIL_SLOT + f, 0)
        ),
        out_shape=jax.ShapeDtypeStruct((TROWS, 2 * EMB), jnp.bfloat16),
        input_output_aliases={3: 0},
    )


def _mm_body(x_ref, w_ref, b_ref, o_ref):
    bm4 = x_ref.shape[0]
    x = x_ref[...].reshape(bm4 // 4, DTOT)
    o_ref[...] = (
        jnp.dot(x, w_ref[...], preferred_element_type=jnp.float32)
        + b_ref[...]
    )


def _mm_body_aliased(x_ref, w_ref, b_ref, _prev_ref, o_ref):
    _mm_body(x_ref, w_ref, b_ref, o_ref)


@functools.lru_cache(maxsize=None)
def _make_matmul(n_tok: int, bm: int, slice_blocks: int, k: int):
    base = k * slice_blocks
    in_specs = [
        pl.BlockSpec((bm * 4, 2 * EMB), lambda i: (i, 0)),
        pl.BlockSpec((DTOT, PROJ), lambda i: (0, 0)),
        pl.BlockSpec((1, PROJ), lambda i: (0, 0)),
    ]
    out_spec = pl.BlockSpec((bm, PROJ), lambda i: (i + base, 0))
    params = pltpu.CompilerParams(skip_device_barrier=True)
    if k == 0:
        return pl.pallas_call(
            _mm_body,
            grid=(slice_blocks,),
            in_specs=in_specs,
            out_specs=out_spec,
            out_shape=jax.ShapeDtypeStruct((n_tok, PROJ), jnp.float32),
            compiler_params=params,
        )
    return pl.pallas_call(
        _mm_body_aliased,
        grid=(slice_blocks,),
        in_specs=in_specs + [pl.BlockSpec(memory_space=pl.ANY)],
        out_specs=out_spec,
        out_shape=jax.ShapeDtypeStruct((n_tok, PROJ), jnp.float32),
        input_output_aliases={3: 0},
        compiler_params=params,
    )


NSLICE = 8
BM = 1600


def kernel(tokens, tables, proj_W, proj_b):
    B, T, F = tokens.shape
    n_tok = B * T
    n_rows = n_tok * F
    tok_s = n_tok // NSLICE
    rows_s = n_rows // NSLICE
    slice_blocks = tok_s // BM

    tt = tables.transpose(0, 2, 1)
    eye = jnp.eye(EMB, dtype=jnp.float32)
    t128 = _make_transpose_main()(tt, eye)
    t128 = _make_transpose_tail()(
        tt[:, :, VMAIN:VMAIN + HTAIL], tt[:, :, VMAIN + HTAIL:], eye, t128
    )
    flat_tables = t128.reshape(2 * TROWS, EMB)

    v = tokens.astype(jnp.int32)
    fld = jnp.arange(F, dtype=jnp.int32)
    m_main = (
        fld * (2 * ROWS_F) + ((v >> 11) << 11) + ((v & 1023) << 1)
        + ((v >> 10) & 1)
    )
    wtl = v - VMAIN
    m_tail = (
        2 * MAIN_ROWS + fld * (2 * TAIL_SLOT) + ((wtl % HTAIL) << 1)
        + (wtl // HTAIL)
    )
    m = jnp.where(v < VMAIN, m_main, m_tail)
    idx = m.reshape(n_rows // CHUNK, CHUNK)

    wt = proj_W.T.astype(jnp.bfloat16)
    b2 = proj_b.reshape(1, PROJ)
    ichunks = rows_s // CHUNK

    out = None
    for k in range(NSLICE):
        g = _make_gather(rows_s, k * ichunks)(flat_tables, idx)
        xk = g.reshape(rows_s // 2, 2 * EMB)
        mm = _make_matmul(n_tok, BM, slice_blocks, k)
        out = mm(xk, wt, b2) if k == 0 else mm(xk, wt, b2, out)
    return out.reshape(B, T, PROJ)


# speedup vs baseline: 2.2375x; 2.2375x over previous
import functools

import jax
import jax.numpy as jnp
from jax import lax
from jax.experimental import pallas as pl
from jax.experimental.pallas import tpu as pltpu
from jax.experimental.pallas import tpu_sc as plsc

NUM_FIELDS = 8
VOCAB = 100000
EMB = 64
PROJ = 512
DTOT = NUM_FIELDS * EMB

NC, NS = 2, 16
NW = NC * NS
CHUNK = 128
NBUF = 5


@functools.lru_cache(maxsize=None)
def _make_gather(n_rows: int, slice_chunk_base: int):
    rows_per_w = n_rows // NW
    chunks_per_w = rows_per_w // CHUNK
    mesh = plsc.VectorSubcoreMesh(core_axis_name="c", subcore_axis_name="s")

    assert chunks_per_w % NBUF == 0
    n_groups = chunks_per_w // NBUF

    @functools.partial(
        pl.kernel,
        mesh=mesh,
        compiler_params=pltpu.CompilerParams(use_tc_tiling_on_sc=False),
        out_type=jax.ShapeDtypeStruct((n_rows, EMB), jnp.float32),
        scratch_types=(
            [pltpu.VMEM((chunks_per_w, CHUNK), jnp.int32)]
            + [pltpu.VMEM((CHUNK, EMB), jnp.float32)] * NBUF
            + [pltpu.SemaphoreType.DMA] * (2 * NBUF)
        ),
    )
    def gather(table_hbm, idx_hbm, out_hbm, idx_v, *scratch):
        wid = lax.axis_index("s") * NC + lax.axis_index("c")
        cbase = slice_chunk_base + wid * chunks_per_w
        rbase = wid * rows_per_w
        bufs = list(scratch[:NBUF])
        gsems = list(scratch[NBUF:2 * NBUF])
        wsems = list(scratch[2 * NBUF:])
        tbl = table_hbm
        pltpu.sync_copy(idx_hbm.at[pl.ds(cbase, chunks_per_w)], idx_v)

        def gather_cp(j, b):
            return pltpu.make_async_copy(
                tbl.at[idx_v.at[j]], bufs[b], gsems[b]
            )

        def wb_cp(j, b):
            return pltpu.make_async_copy(
                bufs[b], out_hbm.at[pl.ds(rbase + j * CHUNK, CHUNK)], wsems[b]
            )

        for b in range(NBUF):
            gather_cp(b, b).start()

        def group(g, _):
            for b in range(NBUF):
                j = g * NBUF + b
                gather_cp(j, b).wait()
                wb_cp(j, b).start()

                @pl.when(g < n_groups - 1)
                def _():
                    wb_cp(j, b).wait()
                    gather_cp(j + NBUF, b).start()
            return _

        lax.fori_loop(0, n_groups, group, None)

        for b in range(NBUF):
            wb_cp(chunks_per_w - NBUF + b, b).wait()

    return gather


VB = 8192
NVB = 98304 // VB
VMAIN = NVB * VB
VTAIL = VOCAB - VMAIN
HTAIL = VTAIL // 2
ROWS_F = VMAIN // 2
MAIN_ROWS = NUM_FIELDS * ROWS_F
TAIL_SLOT = 1024
TROWS = MAIN_ROWS + NUM_FIELDS * TAIL_SLOT


def _mxu_t(x, eye):
    return jax.lax.dot_general(
        x, eye, (((0,), (0,)), ((), ())),
        preferred_element_type=jnp.float32,
    )


def _tr_main_body(x_ref, eye_ref, o_ref):
    x = x_ref[0]
    eye = eye_ref[...]
    o_ref[...] = jnp.concatenate(
        [_mxu_t(x[:, :VB // 2], eye), _mxu_t(x[:, VB // 2:], eye)], axis=1
    )


def _tr_tail_body(xa_ref, xb_ref, eye_ref, _prev_ref, o_ref):
    eye = eye_ref[...]
    t = jnp.concatenate([_mxu_t(xa_ref[0], eye), _mxu_t(xb_ref[0], eye)],
                        axis=1)
    o_ref[...] = jnp.concatenate(
        [t, jnp.zeros((TAIL_SLOT - HTAIL, 2 * EMB), jnp.float32)], axis=0
    )


@functools.lru_cache(maxsize=None)
def _make_transpose_main():
    return pl.pallas_call(
        _tr_main_body,
        grid=(NUM_FIELDS, NVB),
        in_specs=[
            pl.BlockSpec((1, EMB, VB), lambda f, v: (f, 0, v)),
            pl.BlockSpec((EMB, EMB), lambda f, v: (0, 0)),
        ],
        out_specs=pl.BlockSpec(
            (VB // 2, 2 * EMB), lambda f, v: (f * NVB + v, 0)
        ),
        out_shape=jax.ShapeDtypeStruct((TROWS, 2 * EMB), jnp.float32),
    )


@functools.lru_cache(maxsize=None)
def _make_transpose_tail():
    return pl.pallas_call(
        _tr_tail_body,
        grid=(NUM_FIELDS,),
        in_specs=[
            pl.BlockSpec((1, EMB, HTAIL), lambda f: (f, 0, 0)),
            pl.BlockSpec((1, EMB, HTAIL), lambda f: (f, 0, 0)),
            pl.BlockSpec((EMB, EMB), lambda f: (0, 0)),
            pl.BlockSpec(memory_space=pl.ANY),
        ],
        out_specs=pl.BlockSpec(
            (TAIL_SLOT, 2 * EMB), lambda f: (MAIN_ROWS // TAIL_SLOT + f, 0)
        ),
        out_shape=jax.ShapeDtypeStruct((TROWS, 2 * EMB), jnp.float32),
        input_output_aliases={3: 0},
    )


def _mm_body(x_ref, w_ref, b_ref, o_ref):
    bm4 = x_ref.shape[0]
    x = x_ref[...].reshape(bm4 // 4, DTOT)
    o_ref[...] = (
        jnp.dot(x, w_ref[...], preferred_element_type=jnp.float32)
        + b_ref[...]
    )


def _mm_body_aliased(x_ref, w_ref, b_ref, _prev_ref, o_ref):
    _mm_body(x_ref, w_ref, b_ref, o_ref)


@functools.lru_cache(maxsize=None)
def _make_matmul(n_tok: int, bm: int, slice_blocks: int, k: int):
    base = k * slice_blocks
    in_specs = [
        pl.BlockSpec((bm * 4, 2 * EMB), lambda i: (i, 0)),
        pl.BlockSpec((DTOT, PROJ), lambda i: (0, 0)),
        pl.BlockSpec((1, PROJ), lambda i: (0, 0)),
    ]
    out_spec = pl.BlockSpec((bm, PROJ), lambda i: (i + base, 0))
    params = pltpu.CompilerParams(skip_device_barrier=True)
    if k == 0:
        return pl.pallas_call(
            _mm_body,
            grid=(slice_blocks,),
            in_specs=in_specs,
            out_specs=out_spec,
            out_shape=jax.ShapeDtypeStruct((n_tok, PROJ), jnp.float32),
            compiler_params=params,
        )
    return pl.pallas_call(
        _mm_body_aliased,
        grid=(slice_blocks,),
        in_specs=in_specs + [pl.BlockSpec(memory_space=pl.ANY)],
        out_specs=out_spec,
        out_shape=jax.ShapeDtypeStruct((n_tok, PROJ), jnp.float32),
        input_output_aliases={3: 0},
        compiler_params=params,
    )


NSLICE = 8
BM = 1600


def kernel(tokens, tables, proj_W, proj_b):
    B, T, F = tokens.shape
    n_tok = B * T
    n_rows = n_tok * F
    tok_s = n_tok // NSLICE
    rows_s = n_rows // NSLICE
    slice_blocks = tok_s // BM

    tt = tables.transpose(0, 2, 1)
    eye = jnp.eye(EMB, dtype=jnp.float32)
    t128 = _make_transpose_main()(tt, eye)
    t128 = _make_transpose_tail()(
        tt[:, :, VMAIN:VMAIN + HTAIL], tt[:, :, VMAIN + HTAIL:], eye, t128
    )
    flat_tables = t128.reshape(2 * TROWS, EMB)

    v = tokens.astype(jnp.int32)
    fld = jnp.arange(F, dtype=jnp.int32)
    hvb = VB // 2
    m_main = (
        fld * (2 * ROWS_F) + (v // VB) * VB + ((v % hvb) << 1)
        + ((v // hvb) & 1)
    )
    wtl = v - VMAIN
    m_tail = (
        2 * MAIN_ROWS + fld * (2 * TAIL_SLOT) + ((wtl % HTAIL) << 1)
        + (wtl // HTAIL)
    )
    m = jnp.where(v < VMAIN, m_main, m_tail)
    idx = m.reshape(n_rows // CHUNK, CHUNK)

    wt = proj_W.T
    b2 = proj_b.reshape(1, PROJ)
    ichunks = rows_s // CHUNK

    out = None
    for k in range(NSLICE):
        g = _make_gather(rows_s, k * ichunks)(flat_tables, idx)
        xk = g.reshape(rows_s // 2, 2 * EMB)
        mm = _make_matmul(n_tok, BM, slice_blocks, k)
        out = mm(xk, wt, b2) if k == 0 else mm(xk, wt, b2, out)
    return out.reshape(B, T, PROJ)


# speedup vs baseline: 2.3127x; 1.0336x over previous
import functools

import jax
import jax.numpy as jnp
from jax import lax
from jax.experimental import pallas as pl
from jax.experimental.pallas import tpu as pltpu
from jax.experimental.pallas import tpu_sc as plsc

NUM_FIELDS = 8
VOCAB = 100000
EMB = 64
PROJ = 512
DTOT = NUM_FIELDS * EMB

NC, NS = 2, 16
NW = NC * NS
CHUNK = 128
NBUF = 5


@functools.lru_cache(maxsize=None)
def _make_gather(n_rows: int, slice_chunk_base: int):
    rows_per_w = n_rows // NW
    chunks_per_w = rows_per_w // CHUNK
    mesh = plsc.VectorSubcoreMesh(core_axis_name="c", subcore_axis_name="s")

    assert chunks_per_w % NBUF == 0
    n_groups = chunks_per_w // NBUF

    @functools.partial(
        pl.kernel,
        mesh=mesh,
        compiler_params=pltpu.CompilerParams(use_tc_tiling_on_sc=False),
        out_type=jax.ShapeDtypeStruct((n_rows, EMB), jnp.float32),
        scratch_types=(
            [pltpu.VMEM((chunks_per_w, CHUNK), jnp.int32)]
            + [pltpu.VMEM((CHUNK, EMB), jnp.float32)] * NBUF
            + [pltpu.SemaphoreType.DMA] * (2 * NBUF)
        ),
    )
    def gather(table_hbm, idx_hbm, out_hbm, idx_v, *scratch):
        wid = lax.axis_index("s") * NC + lax.axis_index("c")
        cbase = slice_chunk_base + wid * chunks_per_w
        rbase = wid * rows_per_w
        bufs = list(scratch[:NBUF])
        gsems = list(scratch[NBUF:2 * NBUF])
        wsems = list(scratch[2 * NBUF:])
        tbl = table_hbm
        pltpu.sync_copy(idx_hbm.at[pl.ds(cbase, chunks_per_w)], idx_v)

        def gather_cp(j, b):
            return pltpu.make_async_copy(
                tbl.at[idx_v.at[j]], bufs[b], gsems[b]
            )

        def wb_cp(j, b):
            return pltpu.make_async_copy(
                bufs[b], out_hbm.at[pl.ds(rbase + j * CHUNK, CHUNK)], wsems[b]
            )

        for b in range(NBUF):
            gather_cp(b, b).start()

        def group(g, _):
            for b in range(NBUF):
                j = g * NBUF + b
                gather_cp(j, b).wait()
                wb_cp(j, b).start()

                @pl.when(g < n_groups - 1)
                def _():
                    wb_cp(j, b).wait()
                    gather_cp(j + NBUF, b).start()
            return _

        lax.fori_loop(0, n_groups, group, None)

        for b in range(NBUF):
            wb_cp(chunks_per_w - NBUF + b, b).wait()

    return gather


VB = 16384
NVB = 98304 // VB
VMAIN = NVB * VB
VTAIL = VOCAB - VMAIN
HTAIL = VTAIL // 2
ROWS_F = VMAIN // 2
MAIN_ROWS = NUM_FIELDS * ROWS_F
TAIL_SLOT = 1024
TROWS = MAIN_ROWS + NUM_FIELDS * TAIL_SLOT


def _mxu_t(x, eye):
    return jax.lax.dot_general(
        x, eye, (((0,), (0,)), ((), ())),
        preferred_element_type=jnp.float32,
    )


def _tr_main_body(x_ref, eye_ref, o_ref):
    x = x_ref[0]
    eye = eye_ref[...]
    o_ref[...] = jnp.concatenate(
        [_mxu_t(x[:, :VB // 2], eye), _mxu_t(x[:, VB // 2:], eye)], axis=1
    )


def _tr_tail_body(xa_ref, xb_ref, eye_ref, _prev_ref, o_ref):
    eye = eye_ref[...]
    t = jnp.concatenate([_mxu_t(xa_ref[0], eye), _mxu_t(xb_ref[0], eye)],
                        axis=1)
    o_ref[...] = jnp.concatenate(
        [t, jnp.zeros((TAIL_SLOT - HTAIL, 2 * EMB), jnp.float32)], axis=0
    )


@functools.lru_cache(maxsize=None)
def _make_transpose_main():
    return pl.pallas_call(
        _tr_main_body,
        grid=(NUM_FIELDS, NVB),
        in_specs=[
            pl.BlockSpec((1, EMB, VB), lambda f, v: (f, 0, v)),
            pl.BlockSpec((EMB, EMB), lambda f, v: (0, 0)),
        ],
        out_specs=pl.BlockSpec(
            (VB // 2, 2 * EMB), lambda f, v: (f * NVB + v, 0)
        ),
        out_shape=jax.ShapeDtypeStruct((TROWS, 2 * EMB), jnp.float32),
    )


@functools.lru_cache(maxsize=None)
def _make_transpose_tail():
    return pl.pallas_call(
        _tr_tail_body,
        grid=(NUM_FIELDS,),
        in_specs=[
            pl.BlockSpec((1, EMB, HTAIL), lambda f: (f, 0, 0)),
            pl.BlockSpec((1, EMB, HTAIL), lambda f: (f, 0, 0)),
            pl.BlockSpec((EMB, EMB), lambda f: (0, 0)),
            pl.BlockSpec(memory_space=pl.ANY),
        ],
        out_specs=pl.BlockSpec(
            (TAIL_SLOT, 2 * EMB), lambda f: (MAIN_ROWS // TAIL_SLOT + f, 0)
        ),
        out_shape=jax.ShapeDtypeStruct((TROWS, 2 * EMB), jnp.float32),
        input_output_aliases={3: 0},
    )


def _mm_body(x_ref, w_ref, b_ref, o_ref):
    bm4 = x_ref.shape[0]
    x = x_ref[...].reshape(bm4 // 4, DTOT)
    o_ref[...] = (
        jnp.dot(x, w_ref[...], preferred_element_type=jnp.float32)
        + b_ref[...]
    )


def _mm_body_aliased(x_ref, w_ref, b_ref, _prev_ref, o_ref):
    _mm_body(x_ref, w_ref, b_ref, o_ref)


@functools.lru_cache(maxsize=None)
def _make_matmul(n_tok: int, bm: int, slice_blocks: int, k: int):
    base = k * slice_blocks
    in_specs = [
        pl.BlockSpec((bm * 4, 2 * EMB), lambda i: (i, 0)),
        pl.BlockSpec((DTOT, PROJ), lambda i: (0, 0)),
        pl.BlockSpec((1, PROJ), lambda i: (0, 0)),
    ]
    out_spec = pl.BlockSpec((bm, PROJ), lambda i: (i + base, 0))
    params = pltpu.CompilerParams(skip_device_barrier=True)
    if k == 0:
        return pl.pallas_call(
            _mm_body,
            grid=(slice_blocks,),
            in_specs=in_specs,
            out_specs=out_spec,
            out_shape=jax.ShapeDtypeStruct((n_tok, PROJ), jnp.float32),
            compiler_params=params,
        )
    return pl.pallas_call(
        _mm_body_aliased,
        grid=(slice_blocks,),
        in_specs=in_specs + [pl.BlockSpec(memory_space=pl.ANY)],
        out_specs=out_spec,
        out_shape=jax.ShapeDtypeStruct((n_tok, PROJ), jnp.float32),
        input_output_aliases={3: 0},
        compiler_params=params,
    )


NSLICE = 8
BM = 1600


def kernel(tokens, tables, proj_W, proj_b):
    B, T, F = tokens.shape
    n_tok = B * T
    n_rows = n_tok * F
    tok_s = n_tok // NSLICE
    rows_s = n_rows // NSLICE
    slice_blocks = tok_s // BM

    tt = tables.transpose(0, 2, 1)
    eye = jnp.eye(EMB, dtype=jnp.float32)
    t128 = _make_transpose_main()(tt, eye)
    t128 = _make_transpose_tail()(
        tt[:, :, VMAIN:VMAIN + HTAIL], tt[:, :, VMAIN + HTAIL:], eye, t128
    )
    flat_tables = t128.reshape(2 * TROWS, EMB)

    v = tokens.astype(jnp.int32)
    fld = jnp.arange(F, dtype=jnp.int32)
    hvb = VB // 2
    m_main = (
        fld * (2 * ROWS_F) + (v // VB) * VB + ((v % hvb) << 1)
        + ((v // hvb) & 1)
    )
    wtl = v - VMAIN
    m_tail = (
        2 * MAIN_ROWS + fld * (2 * TAIL_SLOT) + ((wtl % HTAIL) << 1)
        + (wtl // HTAIL)
    )
    m = jnp.where(v < VMAIN, m_main, m_tail)
    idx = m.reshape(n_rows // CHUNK, CHUNK)

    wt = proj_W.T
    b2 = proj_b.reshape(1, PROJ)
    ichunks = rows_s // CHUNK

    out = None
    for k in range(NSLICE):
        g = _make_gather(rows_s, k * ichunks)(flat_tables, idx)
        xk = g.reshape(rows_s // 2, 2 * EMB)
        mm = _make_matmul(n_tok, BM, slice_blocks, k)
        out = mm(xk, wt, b2) if k == 0 else mm(xk, wt, b2, out)
    return out.reshape(B, T, PROJ)


# speedup vs baseline: 2.3426x; 1.0129x over previous
import functools

import jax
import jax.numpy as jnp
from jax import lax
from jax.experimental import pallas as pl
from jax.experimental.pallas import tpu as pltpu
from jax.experimental.pallas import tpu_sc as plsc

NUM_FIELDS = 8
VOCAB = 100000
EMB = 64
PROJ = 512
DTOT = NUM_FIELDS * EMB

NC, NS = 2, 16
NW = NC * NS
CHUNK = 128
NBUF = 5


@functools.lru_cache(maxsize=None)
def _make_gather(n_rows: int, slice_chunk_base: int):
    rows_per_w = n_rows // NW
    chunks_per_w = rows_per_w // CHUNK
    mesh = plsc.VectorSubcoreMesh(core_axis_name="c", subcore_axis_name="s")

    assert chunks_per_w % NBUF == 0
    n_groups = chunks_per_w // NBUF

    @functools.partial(
        pl.kernel,
        mesh=mesh,
        compiler_params=pltpu.CompilerParams(use_tc_tiling_on_sc=False),
        out_type=jax.ShapeDtypeStruct((n_rows, EMB), jnp.float32),
        scratch_types=(
            [pltpu.VMEM((chunks_per_w, CHUNK), jnp.int32)]
            + [pltpu.VMEM((CHUNK, EMB), jnp.float32)] * NBUF
            + [pltpu.SemaphoreType.DMA] * (2 * NBUF)
        ),
    )
    def gather(table_hbm, idx_hbm, out_hbm, idx_v, *scratch):
        wid = lax.axis_index("s") * NC + lax.axis_index("c")
        cbase = slice_chunk_base + wid * chunks_per_w
        rbase = wid * rows_per_w
        bufs = list(scratch[:NBUF])
        gsems = list(scratch[NBUF:2 * NBUF])
        wsems = list(scratch[2 * NBUF:])
        tbl = table_hbm
        pltpu.sync_copy(idx_hbm.at[pl.ds(cbase, chunks_per_w)], idx_v)

        def gather_cp(j, b):
            return pltpu.make_async_copy(
                tbl.at[idx_v.at[j]], bufs[b], gsems[b]
            )

        def wb_cp(j, b):
            return pltpu.make_async_copy(
                bufs[b], out_hbm.at[pl.ds(rbase + j * CHUNK, CHUNK)], wsems[b]
            )

        for b in range(NBUF):
            gather_cp(b, b).start()

        def group(g, _):
            for b in range(NBUF):
                j = g * NBUF + b
                gather_cp(j, b).wait()
                wb_cp(j, b).start()

                @pl.when(g < n_groups - 1)
                def _():
                    wb_cp(j, b).wait()
                    gather_cp(j + NBUF, b).start()
            return _

        lax.fori_loop(0, n_groups, group, None)

        for b in range(NBUF):
            wb_cp(chunks_per_w - NBUF + b, b).wait()

    return gather


VB = 32768
NVB = 98304 // VB
VMAIN = NVB * VB
VTAIL = VOCAB - VMAIN
HTAIL = VTAIL // 2
ROWS_F = VMAIN // 2
MAIN_ROWS = NUM_FIELDS * ROWS_F
TAIL_SLOT = 1024
TROWS = MAIN_ROWS + NUM_FIELDS * TAIL_SLOT


def _mxu_t(x, eye):
    return jax.lax.dot_general(
        x, eye, (((0,), (0,)), ((), ())),
        preferred_element_type=jnp.float32,
    )


def _tr_main_body(x_ref, eye_ref, o_ref):
    x = x_ref[0]
    eye = eye_ref[...]
    o_ref[...] = jnp.concatenate(
        [_mxu_t(x[:, :VB // 2], eye), _mxu_t(x[:, VB // 2:], eye)], axis=1
    )


def _tr_tail_body(xa_ref, xb_ref, eye_ref, _prev_ref, o_ref):
    eye = eye_ref[...]
    t = jnp.concatenate([_mxu_t(xa_ref[0], eye), _mxu_t(xb_ref[0], eye)],
                        axis=1)
    o_ref[...] = jnp.concatenate(
        [t, jnp.zeros((TAIL_SLOT - HTAIL, 2 * EMB), jnp.float32)], axis=0
    )


@functools.lru_cache(maxsize=None)
def _make_transpose_main():
    return pl.pallas_call(
        _tr_main_body,
        grid=(NUM_FIELDS, NVB),
        in_specs=[
            pl.BlockSpec((1, EMB, VB), lambda f, v: (f, 0, v)),
            pl.BlockSpec((EMB, EMB), lambda f, v: (0, 0)),
        ],
        out_specs=pl.BlockSpec(
            (VB // 2, 2 * EMB), lambda f, v: (f * NVB + v, 0)
        ),
        out_shape=jax.ShapeDtypeStruct((TROWS, 2 * EMB), jnp.float32),
    )


@functools.lru_cache(maxsize=None)
def _make_transpose_tail():
    return pl.pallas_call(
        _tr_tail_body,
        grid=(NUM_FIELDS,),
        in_specs=[
            pl.BlockSpec((1, EMB, HTAIL), lambda f: (f, 0, 0)),
            pl.BlockSpec((1, EMB, HTAIL), lambda f: (f, 0, 0)),
            pl.BlockSpec((EMB, EMB), lambda f: (0, 0)),
            pl.BlockSpec(memory_space=pl.ANY),
        ],
        out_specs=pl.BlockSpec(
            (TAIL_SLOT, 2 * EMB), lambda f: (MAIN_ROWS // TAIL_SLOT + f, 0)
        ),
        out_shape=jax.ShapeDtypeStruct((TROWS, 2 * EMB), jnp.float32),
        input_output_aliases={3: 0},
    )


def _mm_body(x_ref, w_ref, b_ref, o_ref):
    bm4 = x_ref.shape[0]
    x = x_ref[...].reshape(bm4 // 4, DTOT)
    o_ref[...] = (
        jnp.dot(x, w_ref[...], preferred_element_type=jnp.float32)
        + b_ref[...]
    )


def _mm_body_aliased(x_ref, w_ref, b_ref, _prev_ref, o_ref):
    _mm_body(x_ref, w_ref, b_ref, o_ref)


@functools.lru_cache(maxsize=None)
def _make_matmul(n_tok: int, bm: int, slice_blocks: int, k: int):
    base = k * slice_blocks
    in_specs = [
        pl.BlockSpec((bm * 4, 2 * EMB), lambda i: (i, 0)),
        pl.BlockSpec((DTOT, PROJ), lambda i: (0, 0)),
        pl.BlockSpec((1, PROJ), lambda i: (0, 0)),
    ]
    out_spec = pl.BlockSpec((bm, PROJ), lambda i: (i + base, 0))
    params = pltpu.CompilerParams(skip_device_barrier=True)
    if k == 0:
        return pl.pallas_call(
            _mm_body,
            grid=(slice_blocks,),
            in_specs=in_specs,
            out_specs=out_spec,
            out_shape=jax.ShapeDtypeStruct((n_tok, PROJ), jnp.float32),
            compiler_params=params,
        )
    return pl.pallas_call(
        _mm_body_aliased,
        grid=(slice_blocks,),
        in_specs=in_specs + [pl.BlockSpec(memory_space=pl.ANY)],
        out_specs=out_spec,
        out_shape=jax.ShapeDtypeStruct((n_tok, PROJ), jnp.float32),
        input_output_aliases={3: 0},
        compiler_params=params,
    )


NSLICE = 8
BM = 1600


def kernel(tokens, tables, proj_W, proj_b):
    B, T, F = tokens.shape
    n_tok = B * T
    n_rows = n_tok * F
    tok_s = n_tok // NSLICE
    rows_s = n_rows // NSLICE
    slice_blocks = tok_s // BM

    tt = tables.transpose(0, 2, 1)
    eye = jnp.eye(EMB, dtype=jnp.float32)
    t128 = _make_transpose_main()(tt, eye)
    t128 = _make_transpose_tail()(
        tt[:, :, VMAIN:VMAIN + HTAIL], tt[:, :, VMAIN + HTAIL:], eye, t128
    )
    flat_tables = t128.reshape(2 * TROWS, EMB)

    v = tokens.astype(jnp.int32)
    fld = jnp.arange(F, dtype=jnp.int32)
    hvb = VB // 2
    m_main = (
        fld * (2 * ROWS_F) + (v // VB) * VB + ((v % hvb) << 1)
        + ((v // hvb) & 1)
    )
    wtl = v - VMAIN
    m_tail = (
        2 * MAIN_ROWS + fld * (2 * TAIL_SLOT) + ((wtl % HTAIL) << 1)
        + (wtl // HTAIL)
    )
    m = jnp.where(v < VMAIN, m_main, m_tail)
    idx = m.reshape(n_rows // CHUNK, CHUNK)

    wt = proj_W.T
    b2 = proj_b.reshape(1, PROJ)
    ichunks = rows_s // CHUNK

    out = None
    for k in range(NSLICE):
        g = _make_gather(rows_s, k * ichunks)(flat_tables, idx)
        xk = g.reshape(rows_s // 2, 2 * EMB)
        mm = _make_matmul(n_tok, BM, slice_blocks, k)
        out = mm(xk, wt, b2) if k == 0 else mm(xk, wt, b2, out)
    return out.reshape(B, T, PROJ)


# speedup vs baseline: 2.3442x; 1.0007x over previous
import functools

import jax
import jax.numpy as jnp
from jax import lax
from jax.experimental import pallas as pl
from jax.experimental.pallas import tpu as pltpu
from jax.experimental.pallas import tpu_sc as plsc

NUM_FIELDS = 8
VOCAB = 100000
EMB = 64
PROJ = 512
DTOT = NUM_FIELDS * EMB

NC, NS = 2, 16
NW = NC * NS
CHUNK = 128
NBUF = 5


@functools.lru_cache(maxsize=None)
def _make_gather(n_rows: int, slice_chunk_base: int):
    rows_per_w = n_rows // NW
    chunks_per_w = rows_per_w // CHUNK
    mesh = plsc.VectorSubcoreMesh(core_axis_name="c", subcore_axis_name="s")

    assert chunks_per_w % NBUF == 0
    n_groups = chunks_per_w // NBUF

    @functools.partial(
        pl.kernel,
        mesh=mesh,
        compiler_params=pltpu.CompilerParams(use_tc_tiling_on_sc=False),
        out_type=jax.ShapeDtypeStruct((n_rows, EMB), jnp.float32),
        scratch_types=(
            [pltpu.VMEM((chunks_per_w, CHUNK), jnp.int32)]
            + [pltpu.VMEM((CHUNK, EMB), jnp.float32)] * NBUF
            + [pltpu.SemaphoreType.DMA] * (2 * NBUF)
        ),
    )
    def gather(table_hbm, idx_hbm, out_hbm, idx_v, *scratch):
        wid = lax.axis_index("s") * NC + lax.axis_index("c")
        cbase = slice_chunk_base + wid * chunks_per_w
        rbase = wid * rows_per_w
        bufs = list(scratch[:NBUF])
        gsems = list(scratch[NBUF:2 * NBUF])
        wsems = list(scratch[2 * NBUF:])
        tbl = table_hbm
        pltpu.sync_copy(idx_hbm.at[pl.ds(cbase, chunks_per_w)], idx_v)

        def gather_cp(j, b):
            return pltpu.make_async_copy(
                tbl.at[idx_v.at[j]], bufs[b], gsems[b]
            )

        def wb_cp(j, b):
            return pltpu.make_async_copy(
                bufs[b], out_hbm.at[pl.ds(rbase + j * CHUNK, CHUNK)], wsems[b]
            )

        for b in range(NBUF):
            gather_cp(b, b).start()

        def group(g, _):
            for b in range(NBUF):
                j = g * NBUF + b
                gather_cp(j, b).wait()
                wb_cp(j, b).start()

                @pl.when(g < n_groups - 1)
                def _():
                    wb_cp(j, b).wait()
                    gather_cp(j + NBUF, b).start()
            return _

        lax.fori_loop(0, n_groups, group, None)

        for b in range(NBUF):
            wb_cp(chunks_per_w - NBUF + b, b).wait()

    return gather


VB = 32768
NVB = 98304 // VB
VMAIN = NVB * VB
VTAIL = VOCAB - VMAIN
HTAIL = VTAIL // 2
ROWS_F = VMAIN // 2
MAIN_ROWS = NUM_FIELDS * ROWS_F
TAIL_SLOT = 1024
TROWS = MAIN_ROWS + NUM_FIELDS * TAIL_SLOT


def _mxu_t(x, eye):
    return jax.lax.dot_general(
        x, eye, (((0,), (0,)), ((), ())),
        preferred_element_type=jnp.float32,
    )


def _tr_main_body(x_ref, eye_ref, o_ref):
    x = x_ref[0]
    eye = eye_ref[...]
    o_ref[...] = jnp.concatenate(
        [_mxu_t(x[:, :VB // 2], eye), _mxu_t(x[:, VB // 2:], eye)], axis=1
    )


def _tr_tail_body(xa_ref, xb_ref, eye_ref, _prev_ref, o_ref):
    eye = eye_ref[...]
    t = jnp.concatenate([_mxu_t(xa_ref[0], eye), _mxu_t(xb_ref[0], eye)],
                        axis=1)
    o_ref[...] = jnp.concatenate(
        [t, jnp.zeros((TAIL_SLOT - HTAIL, 2 * EMB), jnp.float32)], axis=0
    )


@functools.lru_cache(maxsize=None)
def _make_transpose_main():
    return pl.pallas_call(
        _tr_main_body,
        grid=(NUM_FIELDS, NVB),
        in_specs=[
            pl.BlockSpec((1, EMB, VB), lambda f, v: (f, 0, v)),
            pl.BlockSpec((EMB, EMB), lambda f, v: (0, 0)),
        ],
        out_specs=pl.BlockSpec(
            (VB // 2, 2 * EMB), lambda f, v: (f * NVB + v, 0)
        ),
        out_shape=jax.ShapeDtypeStruct((TROWS, 2 * EMB), jnp.float32),
    )


@functools.lru_cache(maxsize=None)
def _make_transpose_tail():
    return pl.pallas_call(
        _tr_tail_body,
        grid=(NUM_FIELDS,),
        in_specs=[
            pl.BlockSpec((1, EMB, HTAIL), lambda f: (f, 0, 0)),
            pl.BlockSpec((1, EMB, HTAIL), lambda f: (f, 0, 0)),
            pl.BlockSpec((EMB, EMB), lambda f: (0, 0)),
            pl.BlockSpec(memory_space=pl.ANY),
        ],
        out_specs=pl.BlockSpec(
            (TAIL_SLOT, 2 * EMB), lambda f: (MAIN_ROWS // TAIL_SLOT + f, 0)
        ),
        out_shape=jax.ShapeDtypeStruct((TROWS, 2 * EMB), jnp.float32),
        input_output_aliases={3: 0},
    )


def _mm_body(x_ref, w_ref, b_ref, o_ref):
    bm4 = x_ref.shape[0]
    x = x_ref[...].reshape(bm4 // 4, DTOT)
    o_ref[...] = (
        jnp.dot(x, w_ref[...], preferred_element_type=jnp.float32)
        + b_ref[...]
    )


def _mm_body_aliased(x_ref, w_ref, b_ref, _prev_ref, o_ref):
    _mm_body(x_ref, w_ref, b_ref, o_ref)


@functools.lru_cache(maxsize=None)
def _make_matmul(n_tok: int, bm: int, slice_blocks: int, k: int):
    base = k * slice_blocks
    in_specs = [
        pl.BlockSpec((bm * 4, 2 * EMB), lambda i: (i, 0)),
        pl.BlockSpec((DTOT, PROJ), lambda i: (0, 0)),
        pl.BlockSpec((1, PROJ), lambda i: (0, 0)),
    ]
    out_spec = pl.BlockSpec((bm, PROJ), lambda i: (i + base, 0))
    params = pltpu.CompilerParams(skip_device_barrier=True)
    if k == 0:
        return pl.pallas_call(
            _mm_body,
            grid=(slice_blocks,),
            in_specs=in_specs,
            out_specs=out_spec,
            out_shape=jax.ShapeDtypeStruct((n_tok, PROJ), jnp.float32),
            compiler_params=params,
        )
    return pl.pallas_call(
        _mm_body_aliased,
        grid=(slice_blocks,),
        in_specs=in_specs + [pl.BlockSpec(memory_space=pl.ANY)],
        out_specs=out_spec,
        out_shape=jax.ShapeDtypeStruct((n_tok, PROJ), jnp.float32),
        input_output_aliases={3: 0},
        compiler_params=params,
    )


NSLICE = 8
BM = 1600


def kernel(tokens, tables, proj_W, proj_b):
    B, T, F = tokens.shape
    n_tok = B * T
    n_rows = n_tok * F
    tok_s = n_tok // NSLICE
    rows_s = n_rows // NSLICE
    slice_blocks = tok_s // BM

    tt = tables.transpose(0, 2, 1)
    eye = jnp.eye(EMB, dtype=jnp.float32)
    t128 = _make_transpose_main()(tt, eye)
    t128 = _make_transpose_tail()(
        tt[:, :, VMAIN:VMAIN + HTAIL], tt[:, :, VMAIN + HTAIL:], eye, t128
    )
    flat_tables = lax.optimization_barrier(t128.reshape(-1)).reshape(
        2 * TROWS, EMB
    )

    v = tokens.astype(jnp.int32)
    fld = jnp.arange(F, dtype=jnp.int32)
    hvb = VB // 2
    m_main = (
        fld * (2 * ROWS_F) + (v // VB) * VB + ((v % hvb) << 1)
        + ((v // hvb) & 1)
    )
    wtl = v - VMAIN
    m_tail = (
        2 * MAIN_ROWS + fld * (2 * TAIL_SLOT) + ((wtl % HTAIL) << 1)
        + (wtl // HTAIL)
    )
    m = jnp.where(v < VMAIN, m_main, m_tail)
    idx = m.reshape(n_rows // CHUNK, CHUNK)

    wt = proj_W.T
    b2 = proj_b.reshape(1, PROJ)
    ichunks = rows_s // CHUNK

    out = None
    for k in range(NSLICE):
        g = _make_gather(rows_s, k * ichunks)(flat_tables, idx)
        xk = g.reshape(rows_s // 2, 2 * EMB)
        mm = _make_matmul(n_tok, BM, slice_blocks, k)
        out = mm(xk, wt, b2) if k == 0 else mm(xk, wt, b2, out)
    return out.reshape(B, T, PROJ)
